# baseline (speedup 1.0000x reference)
import jax
import jax.numpy as jnp
from jax import lax
from jax.experimental import pallas as pl
from jax.experimental.pallas import tpu as pltpu
from jax.experimental.pallas import tpu_sc as plsc

N = 10000
E = 320000
D = 128
HID = 128
HEADS = 8
OUT = 128

NW = 32
EW = 10240
NCHUNK = 80
CHUNK = 128
NP = N + 240
ROWS_PER_TILE = N // 16

_f32 = jnp.float32
_i32 = jnp.int32


def _splat(v, h):
    idx = jnp.full((16, 1), h, dtype=_i32)
    dnums = lax.GatherDimensionNumbers(
        offset_dims=(), collapsed_slice_dims=(0,), start_index_map=(0,))
    return lax.gather(v, idx, dnums, (1,),
                      mode=lax.GatherScatterMode.PROMISE_IN_BOUNDS)


def _leaky_exp(s):
    return jnp.exp(jnp.maximum(s, 0.0) + 0.2 * jnp.minimum(s, 0.0))


def _edge_body(is_layer2):
    def body(edges_hbm, atab_hbm, atabsw_hbm, xw_hbm, acc_out, den_out,
             sd_a, sd_b, ats_a, ats_b, atd_a, atd_b, ex_a, ex_b, xw_a, xw_b,
             acc_sh, den_sh, gsem_a, gsem_b, ssem_a, ssem_b):
        cid = lax.axis_index("c")
        sid = lax.axis_index("s")
        w = cid * 16 + sid
        cbase = w * NCHUNK
        col = lax.iota(_i32, 16)
        col8 = lax.bitwise_and(col, 7)
        m8 = col < 8
        cols = [col + h * 16 for h in range(8)]
        zero16 = jnp.zeros((16,), _f32)

        @pl.loop(0, 25)
        def _(r):
            row = jnp.full((16,), r, _i32)
            for cb in range(8):
                plsc.store_scatter(xw_a, [row, cols[cb]], zero16)
            plsc.store_scatter(ex_a, [row, col8], zero16, mask=m8)

        base = sid * ROWS_PER_TILE

        @pl.loop(0, 25)
        def _(k):
            pltpu.sync_copy(xw_a.at[pl.ds(0, 25)],
                            acc_sh.at[pl.ds(base + k * 25, 25)])
            pltpu.sync_copy(ex_a.at[pl.ds(0, 25)],
                            den_sh.at[pl.ds(base + k * 25, 25)])

        plsc.subcore_barrier()

        def issue_gathers(c, sd_v, ats_v, atd_v, xw_v, sem):
            pltpu.sync_copy(edges_hbm.at[c], sd_v)
            pltpu.async_copy(atab_hbm.at[sd_v.at[0]], ats_v, sem)
            pltpu.async_copy(atabsw_hbm.at[sd_v.at[1]], atd_v, sem)
            pltpu.async_copy(xw_hbm.at[sd_v.at[0]], xw_v, sem)

        def wait_gathers(sd_v, ats_v, atd_v, xw_v, sem):
            pltpu.make_async_copy(atab_hbm.at[sd_v.at[0]], ats_v, sem).wait()
            pltpu.make_async_copy(atabsw_hbm.at[sd_v.at[1]], atd_v, sem).wait()
            pltpu.make_async_copy(xw_hbm.at[sd_v.at[0]], xw_v, sem).wait()

        def issue_scatters(sd_v, ex_v, xw_v, sem):
            pltpu.async_copy(xw_v, acc_sh.at[sd_v.at[1]], sem, add=True)
            pltpu.async_copy(ex_v, den_sh.at[sd_v.at[1]], sem, add=True)

        def wait_scatters(sd_v, ex_v, xw_v, sem):
            pltpu.make_async_copy(xw_v, acc_sh.at[sd_v.at[1]], sem).wait()
            pltpu.make_async_copy(ex_v, den_sh.at[sd_v.at[1]], sem).wait()

        if not is_layer2:
            def compute(ats_v, atd_v, ex_v, xw_v):
                @plsc.parallel_loop(0, CHUNK, unroll=6)
                def _(e):
                    row = jnp.full((16,), e, _i32)
                    va = plsc.load_gather(ats_v, [row, col])
                    vb = plsc.load_gather(atd_v, [row, col])
                    exr = _leaky_exp(va + vb)
                    plsc.store_scatter(ex_v, [row, col8], exr, mask=m8)
                    for h in range(HEADS):
                        sp = _splat(exr, h)
                        xv = plsc.load_gather(xw_v, [row, cols[h]])
                        plsc.store_scatter(xw_v, [row, cols[h]], xv * sp)
        else:
            def compute(ats_v, atd_v, ex_v, xw_v):
                @plsc.parallel_loop(0, CHUNK, unroll=6)
                def _(e):
                    row = jnp.full((16,), e, _i32)
                    va = plsc.load_gather(ats_v, [row])
                    vb = plsc.load_gather(atd_v, [row])
                    sp = _leaky_exp(va + vb)
                    plsc.store_scatter(ex_v, [row, col8], sp, mask=m8)
                    for h in range(8):
                        xv = plsc.load_gather(xw_v, [row, cols[h]])
                        plsc.store_scatter(xw_v, [row, cols[h]], xv * sp)

        issue_gathers(cbase, sd_a, ats_a, atd_a, xw_a, gsem_a)

        @pl.loop(0, NCHUNK // 2)
        def _(t):
            c0 = cbase + 2 * t

            @pl.when(t > 0)
            def _():
                wait_scatters(sd_b, ex_b, xw_b, ssem_b)

            issue_gathers(c0 + 1, sd_b, ats_b, atd_b, xw_b, gsem_b)
            wait_gathers(sd_a, ats_a, atd_a, xw_a, gsem_a)
            compute(ats_a, atd_a, ex_a, xw_a)
            issue_scatters(sd_a, ex_a, xw_a, ssem_a)

            wait_scatters(sd_a, ex_a, xw_a, ssem_a)
            c2 = jnp.minimum(c0 + 2, cbase + NCHUNK - 1)
            issue_gathers(c2, sd_a, ats_a, atd_a, xw_a, gsem_a)
            wait_gathers(sd_b, ats_b, atd_b, xw_b, gsem_b)
            compute(ats_b, atd_b, ex_b, xw_b)
            issue_scatters(sd_b, ex_b, xw_b, ssem_b)

        wait_gathers(sd_a, ats_a, atd_a, xw_a, gsem_a)
        wait_scatters(sd_b, ex_b, xw_b, ssem_b)

        plsc.subcore_barrier()
        pltpu.sync_copy(acc_sh.at[pl.ds(base, ROWS_PER_TILE)],
                        acc_out.at[cid, pl.ds(base, ROWS_PER_TILE)])
        pltpu.sync_copy(den_sh.at[pl.ds(base, ROWS_PER_TILE)],
                        den_out.at[cid, pl.ds(base, ROWS_PER_TILE)])

    return body


def _make_sc(is_layer2):
    mesh = plsc.VectorSubcoreMesh(core_axis_name="c", subcore_axis_name="s")
    a_shape = (CHUNK,) if is_layer2 else (CHUNK, 16)
    scratch = [
        pltpu.VMEM((2, CHUNK), _i32),
        pltpu.VMEM((2, CHUNK), _i32),
        pltpu.VMEM(a_shape, _f32),
        pltpu.VMEM(a_shape, _f32),
        pltpu.VMEM(a_shape, _f32),
        pltpu.VMEM(a_shape, _f32),
        pltpu.VMEM((CHUNK, 8), _f32),
        pltpu.VMEM((CHUNK, 8), _f32),
        pltpu.VMEM((CHUNK, 128), _f32),
        pltpu.VMEM((CHUNK, 128), _f32),
        pltpu.VMEM_SHARED((N, 128), _f32),
        pltpu.VMEM_SHARED((N, 8), _f32),
        pltpu.SemaphoreType.DMA,
        pltpu.SemaphoreType.DMA,
        pltpu.SemaphoreType.DMA,
        pltpu.SemaphoreType.DMA,
    ]
    return pl.kernel(
        _edge_body(is_layer2),
        mesh=mesh,
        out_type=[
            jax.ShapeDtypeStruct((2, N, 128), _f32),
            jax.ShapeDtypeStruct((2, N, 8), _f32),
        ],
        scratch_types=scratch,
        compiler_params=pltpu.CompilerParams(
            use_tc_tiling_on_sc=False, needs_layout_passes=False),
    )


def _tc1_body(x_ref, w1_ref, a1_ref, xw_ref, atab_ref, atabsw_ref):
    xw = jnp.dot(x_ref[...], w1_ref[...], preferred_element_type=_f32)
    xw_ref[pl.ds(0, N), :] = xw
    xw_ref[pl.ds(N, NP - N), :] = jnp.zeros((NP - N, HID), _f32)
    at = jnp.dot(xw, a1_ref[...], preferred_element_type=_f32)
    atab_ref[pl.ds(0, N), :] = at[:, :16]
    atab_ref[pl.ds(N, NP - N), :] = jnp.full((NP - N, 16), -1e30, _f32)
    atabsw_ref[pl.ds(0, N), :] = at[:, 16:]
    atabsw_ref[pl.ds(N, NP - N), :] = jnp.full((NP - N, 16), -1e30, _f32)


def _tc2_body(acc_ref, den_ref, b8_ref, b1_ref, w2_ref, a2_ref,
              xw2_ref, atab2_ref):
    dens = den_ref[0] + den_ref[1]
    den128 = jnp.dot(dens, b8_ref[...], preferred_element_type=_f32)
    out1 = (acc_ref[0] + acc_ref[1]) / (den128 + 1e-16) + b1_ref[...]
    h = jnp.where(out1 > 0, out1, jnp.exp(out1) - 1.0)
    xw2 = jnp.dot(h, w2_ref[...], preferred_element_type=_f32)
    xw2_ref[pl.ds(0, N), :] = xw2
    xw2_ref[pl.ds(N, NP - N), :] = jnp.zeros((NP - N, OUT), _f32)
    at2 = jnp.dot(xw2, a2_ref[...], preferred_element_type=_f32)
    atab2_ref[pl.ds(0, N), :] = at2
    atab2_ref[pl.ds(N, NP - N), :] = jnp.full((NP - N, 2), -1e30, _f32)


def _tc3_body(acc_ref, den_ref, bsel_ref, b2_ref, o_ref):
    dens = den_ref[0] + den_ref[1]
    den128 = jnp.dot(dens, bsel_ref[...], preferred_element_type=_f32)
    o_ref[...] = (acc_ref[0] + acc_ref[1]) / (den128 + 1e-16) + b2_ref[...]


def kernel(x, edge_index, W1, att_src1, att_dst1, bias1, W2, att_src2,
           att_dst2, bias2):
    src = edge_index[0]
    dst = edge_index[1]

    pad_src = N + jnp.arange(240, dtype=_i32)
    pad_dst = (jnp.arange(240, dtype=_i32) * 41) % N
    src_p = jnp.concatenate(
        [src.reshape(NW, E // NW),
         jnp.broadcast_to(pad_src, (NW, 240))], axis=1)
    dst_p = jnp.concatenate(
        [dst.reshape(NW, E // NW),
         jnp.broadcast_to(pad_dst, (NW, 240))], axis=1)
    edges = jnp.stack(
        [src_p.reshape(NW * NCHUNK, CHUNK),
         dst_p.reshape(NW * NCHUNK, CHUNK)], axis=1)

    a_s = att_src1.reshape(HEADS, HID // HEADS)
    a_d = att_dst1.reshape(HEADS, HID // HEADS)
    eye8 = jnp.eye(HEADS, dtype=_f32)
    A1s = (eye8[:, None, :] * a_s[:, :, None]).reshape(HID, HEADS)
    A1d = (eye8[:, None, :] * a_d[:, :, None]).reshape(HID, HEADS)
    A1cat = jnp.concatenate([A1s, A1d, A1d, A1s], axis=1)
    A2cat = jnp.concatenate(
        [att_src2.reshape(OUT, 1), att_dst2.reshape(OUT, 1)], axis=1)

    lane_h = jnp.arange(128, dtype=_i32) // 16
    B8 = (jnp.arange(8, dtype=_i32)[:, None] == lane_h[None, :]).astype(_f32)
    B1sel = (jnp.arange(8, dtype=_i32)[:, None] == 0).astype(_f32)
    B1sel = jnp.broadcast_to(B1sel, (8, 128)).astype(_f32)

    xw1, atab, atabsw = pl.pallas_call(
        _tc1_body,
        out_shape=[
            jax.ShapeDtypeStruct((NP, HID), _f32),
            jax.ShapeDtypeStruct((NP, 16), _f32),
            jax.ShapeDtypeStruct((NP, 16), _f32),
        ],
    )(x, W1, A1cat)

    acc1, den1 = _make_sc(is_layer2=False)(edges, atab, atabsw, xw1)

    xw2, atab2 = pl.pallas_call(
        _tc2_body,
        out_shape=[
            jax.ShapeDtypeStruct((NP, OUT), _f32),
            jax.ShapeDtypeStruct((NP, 2), _f32),
        ],
    )(acc1, den1, B8, bias1.reshape(1, HID), W2, A2cat)

    as2 = atab2[:, 0]
    ad2 = atab2[:, 1]

    acc2, den2 = _make_sc(is_layer2=True)(edges, as2, ad2, xw2)

    out = pl.pallas_call(
        _tc3_body,
        out_shape=jax.ShapeDtypeStruct((N, OUT), _f32),
    )(acc2, den2, B1sel, bias2.reshape(1, OUT))

    return out


# speedup vs baseline: 1.0260x; 1.0260x over previous
import jax
import jax.numpy as jnp
from jax import lax
from jax.experimental import pallas as pl
from jax.experimental.pallas import tpu as pltpu
from jax.experimental.pallas import tpu_sc as plsc

N = 10000
E = 320000
D = 128
HID = 128
HEADS = 8
OUT = 128

NW = 32
EW = 10240
NCHUNK = 80
CHUNK = 128
NP = N + 240
ROWS_PER_TILE = N // 16

_f32 = jnp.float32
_i32 = jnp.int32


def _splat(v, h):
    idx = jnp.full((16, 1), h, dtype=_i32)
    dnums = lax.GatherDimensionNumbers(
        offset_dims=(), collapsed_slice_dims=(0,), start_index_map=(0,))
    return lax.gather(v, idx, dnums, (1,),
                      mode=lax.GatherScatterMode.PROMISE_IN_BOUNDS)


def _leaky_exp(s):
    return jnp.exp(jnp.maximum(s, 0.0) + 0.2 * jnp.minimum(s, 0.0))


def _edge_body(is_layer2):
    def body(edges_hbm, atab_hbm, atabsw_hbm, xw_hbm, acc_out, den_out,
             sd_a, sd_b, ats_a, ats_b, atd_a, atd_b, ex_a, ex_b, xw_a, xw_b,
             acc_sh, den_sh, gsem_a, gsem_b, ssem_a, ssem_b):
        cid = lax.axis_index("c")
        sid = lax.axis_index("s")
        w = cid * 16 + sid
        cbase = w * NCHUNK
        col = lax.iota(_i32, 16)
        col8 = lax.bitwise_and(col, 7)
        m8 = col < 8
        cols = [col + h * 16 for h in range(8)]
        zero16 = jnp.zeros((16,), _f32)

        @pl.loop(0, 25)
        def _(r):
            row = jnp.full((16,), r, _i32)
            for cb in range(8):
                plsc.store_scatter(xw_a, [row, cols[cb]], zero16)
            plsc.store_scatter(ex_a, [row, col8], zero16, mask=m8)

        base = sid * ROWS_PER_TILE

        @pl.loop(0, 25)
        def _(k):
            pltpu.sync_copy(xw_a.at[pl.ds(0, 25)],
                            acc_sh.at[pl.ds(base + k * 25, 25)])
            pltpu.sync_copy(ex_a.at[pl.ds(0, 25)],
                            den_sh.at[pl.ds(base + k * 25, 25)])

        plsc.subcore_barrier()

        def issue_gathers(c, sd_v, ats_v, atd_v, xw_v, sem):
            pltpu.sync_copy(edges_hbm.at[c], sd_v)
            pltpu.async_copy(atab_hbm.at[sd_v.at[0]], ats_v, sem)
            pltpu.async_copy(atabsw_hbm.at[sd_v.at[1]], atd_v, sem)
            pltpu.async_copy(xw_hbm.at[sd_v.at[0]], xw_v, sem)

        def wait_gathers(sd_v, ats_v, atd_v, xw_v, sem):
            pltpu.make_async_copy(atab_hbm.at[sd_v.at[0]], ats_v, sem).wait()
            pltpu.make_async_copy(atabsw_hbm.at[sd_v.at[1]], atd_v, sem).wait()
            pltpu.make_async_copy(xw_hbm.at[sd_v.at[0]], xw_v, sem).wait()

        def issue_scatters(sd_v, ex_v, xw_v, sem):
            pltpu.async_copy(xw_v, acc_sh.at[sd_v.at[1]], sem, add=True)
            pltpu.async_copy(ex_v, den_sh.at[sd_v.at[1]], sem, add=True)

        def wait_scatters(sd_v, ex_v, xw_v, sem):
            pltpu.make_async_copy(xw_v, acc_sh.at[sd_v.at[1]], sem).wait()
            pltpu.make_async_copy(ex_v, den_sh.at[sd_v.at[1]], sem).wait()

        if not is_layer2:
            def compute(ats_v, atd_v, ex_v, xw_v):
                @plsc.parallel_loop(0, CHUNK, unroll=6)
                def _(e):
                    row = jnp.full((16,), e, _i32)
                    va = plsc.load_gather(ats_v, [row, col])
                    vb = plsc.load_gather(atd_v, [row, col])
                    exr = _leaky_exp(va + vb)
                    plsc.store_scatter(ex_v, [row, col8], exr, mask=m8)
                    for h in range(HEADS):
                        sp = _splat(exr, h)
                        xv = plsc.load_gather(xw_v, [row, cols[h]])
                        plsc.store_scatter(xw_v, [row, cols[h]], xv * sp)
        else:
            def compute(ats_v, atd_v, ex_v, xw_v):
                @plsc.parallel_loop(0, CHUNK, unroll=4)
                def _(e):
                    row = jnp.full((16,), e, _i32)
                    va = plsc.load_gather(ats_v, [row])
                    vb = plsc.load_gather(atd_v, [row])
                    sp = _leaky_exp(va + vb)
                    plsc.store_scatter(ex_v, [row, col8], sp, mask=m8)
                    for h in range(8):
                        xv = plsc.load_gather(xw_v, [row, cols[h]])
                        plsc.store_scatter(xw_v, [row, cols[h]], xv * sp)

        issue_gathers(cbase, sd_a, ats_a, atd_a, xw_a, gsem_a)

        @pl.loop(0, NCHUNK // 2)
        def _(t):
            c0 = cbase + 2 * t

            @pl.when(t > 0)
            def _():
                wait_scatters(sd_b, ex_b, xw_b, ssem_b)

            issue_gathers(c0 + 1, sd_b, ats_b, atd_b, xw_b, gsem_b)
            wait_gathers(sd_a, ats_a, atd_a, xw_a, gsem_a)
            compute(ats_a, atd_a, ex_a, xw_a)
            issue_scatters(sd_a, ex_a, xw_a, ssem_a)

            wait_scatters(sd_a, ex_a, xw_a, ssem_a)
            c2 = jnp.minimum(c0 + 2, cbase + NCHUNK - 1)
            issue_gathers(c2, sd_a, ats_a, atd_a, xw_a, gsem_a)
            wait_gathers(sd_b, ats_b, atd_b, xw_b, gsem_b)
            compute(ats_b, atd_b, ex_b, xw_b)
            issue_scatters(sd_b, ex_b, xw_b, ssem_b)

        wait_gathers(sd_a, ats_a, atd_a, xw_a, gsem_a)
        wait_scatters(sd_b, ex_b, xw_b, ssem_b)

        plsc.subcore_barrier()
        pltpu.sync_copy(acc_sh.at[pl.ds(base, ROWS_PER_TILE)],
                        acc_out.at[cid, pl.ds(base, ROWS_PER_TILE)])
        pltpu.sync_copy(den_sh.at[pl.ds(base, ROWS_PER_TILE)],
                        den_out.at[cid, pl.ds(base, ROWS_PER_TILE)])

    return body


def _make_sc(is_layer2):
    mesh = plsc.VectorSubcoreMesh(core_axis_name="c", subcore_axis_name="s")
    a_shape = (CHUNK,) if is_layer2 else (CHUNK, 16)
    scratch = [
        pltpu.VMEM((2, CHUNK), _i32),
        pltpu.VMEM((2, CHUNK), _i32),
        pltpu.VMEM(a_shape, _f32),
        pltpu.VMEM(a_shape, _f32),
        pltpu.VMEM(a_shape, _f32),
        pltpu.VMEM(a_shape, _f32),
        pltpu.VMEM((CHUNK, 8), _f32),
        pltpu.VMEM((CHUNK, 8), _f32),
        pltpu.VMEM((CHUNK, 128), _f32),
        pltpu.VMEM((CHUNK, 128), _f32),
        pltpu.VMEM_SHARED((N, 128), _f32),
        pltpu.VMEM_SHARED((N, 8), _f32),
        pltpu.SemaphoreType.DMA,
        pltpu.SemaphoreType.DMA,
        pltpu.SemaphoreType.DMA,
        pltpu.SemaphoreType.DMA,
    ]
    return pl.kernel(
        _edge_body(is_layer2),
        mesh=mesh,
        out_type=[
            jax.ShapeDtypeStruct((2, N, 128), _f32),
            jax.ShapeDtypeStruct((2, N, 8), _f32),
        ],
        scratch_types=scratch,
        compiler_params=pltpu.CompilerParams(
            use_tc_tiling_on_sc=False, needs_layout_passes=False),
    )


def _tc1_body(x_ref, w1_ref, a1_ref, xw_ref, atab_ref, atabsw_ref):
    xw = jnp.dot(x_ref[...], w1_ref[...], preferred_element_type=_f32)
    xw_ref[pl.ds(0, N), :] = xw
    xw_ref[pl.ds(N, NP - N), :] = jnp.zeros((NP - N, HID), _f32)
    at = jnp.dot(xw, a1_ref[...], preferred_element_type=_f32)
    atab_ref[pl.ds(0, N), :] = at[:, :16]
    atab_ref[pl.ds(N, NP - N), :] = jnp.full((NP - N, 16), -1e30, _f32)
    atabsw_ref[pl.ds(0, N), :] = at[:, 16:]
    atabsw_ref[pl.ds(N, NP - N), :] = jnp.full((NP - N, 16), -1e30, _f32)


def _tc2_body(acc_ref, den_ref, b8_ref, b1_ref, w2_ref, a2_ref,
              xw2_ref, atab2_ref):
    dens = den_ref[0] + den_ref[1]
    den128 = jnp.dot(dens, b8_ref[...], preferred_element_type=_f32)
    out1 = (acc_ref[0] + acc_ref[1]) / (den128 + 1e-16) + b1_ref[...]
    h = jnp.where(out1 > 0, out1, jnp.exp(out1) - 1.0)
    xw2 = jnp.dot(h, w2_ref[...], preferred_element_type=_f32)
    xw2_ref[pl.ds(0, N), :] = xw2
    xw2_ref[pl.ds(N, NP - N), :] = jnp.zeros((NP - N, OUT), _f32)
    at2 = jnp.dot(xw2, a2_ref[...], preferred_element_type=_f32)
    atab2_ref[pl.ds(0, N), :] = at2
    atab2_ref[pl.ds(N, NP - N), :] = jnp.full((NP - N, 2), -1e30, _f32)


def _tc3_body(acc_ref, den_ref, bsel_ref, b2_ref, o_ref):
    dens = den_ref[0] + den_ref[1]
    den128 = jnp.dot(dens, bsel_ref[...], preferred_element_type=_f32)
    o_ref[...] = (acc_ref[0] + acc_ref[1]) / (den128 + 1e-16) + b2_ref[...]


def kernel(x, edge_index, W1, att_src1, att_dst1, bias1, W2, att_src2,
           att_dst2, bias2):
    src = edge_index[0]
    dst = edge_index[1]

    pad_src = N + jnp.arange(240, dtype=_i32)
    pad_dst = (jnp.arange(240, dtype=_i32) * 41) % N
    src_p = jnp.concatenate(
        [src.reshape(NW, E // NW),
         jnp.broadcast_to(pad_src, (NW, 240))], axis=1)
    dst_p = jnp.concatenate(
        [dst.reshape(NW, E // NW),
         jnp.broadcast_to(pad_dst, (NW, 240))], axis=1)
    edges = jnp.stack(
        [src_p.reshape(NW * NCHUNK, CHUNK),
         dst_p.reshape(NW * NCHUNK, CHUNK)], axis=1)

    a_s = att_src1.reshape(HEADS, HID // HEADS)
    a_d = att_dst1.reshape(HEADS, HID // HEADS)
    eye8 = jnp.eye(HEADS, dtype=_f32)
    A1s = (eye8[:, None, :] * a_s[:, :, None]).reshape(HID, HEADS)
    A1d = (eye8[:, None, :] * a_d[:, :, None]).reshape(HID, HEADS)
    A1cat = jnp.concatenate([A1s, A1d, A1d, A1s], axis=1)
    A2cat = jnp.concatenate(
        [att_src2.reshape(OUT, 1), att_dst2.reshape(OUT, 1)], axis=1)

    lane_h = jnp.arange(128, dtype=_i32) // 16
    B8 = (jnp.arange(8, dtype=_i32)[:, None] == lane_h[None, :]).astype(_f32)
    B1sel = (jnp.arange(8, dtype=_i32)[:, None] == 0).astype(_f32)
    B1sel = jnp.broadcast_to(B1sel, (8, 128)).astype(_f32)

    xw1, atab, atabsw = pl.pallas_call(
        _tc1_body,
        out_shape=[
            jax.ShapeDtypeStruct((NP, HID), _f32),
            jax.ShapeDtypeStruct((NP, 16), _f32),
            jax.ShapeDtypeStruct((NP, 16), _f32),
        ],
    )(x, W1, A1cat)

    acc1, den1 = _make_sc(is_layer2=False)(edges, atab, atabsw, xw1)

    xw2, atab2 = pl.pallas_call(
        _tc2_body,
        out_shape=[
            jax.ShapeDtypeStruct((NP, OUT), _f32),
            jax.ShapeDtypeStruct((NP, 2), _f32),
        ],
    )(acc1, den1, B8, bias1.reshape(1, HID), W2, A2cat)

    as2 = atab2[:, 0]
    ad2 = atab2[:, 1]

    acc2, den2 = _make_sc(is_layer2=True)(edges, as2, ad2, xw2)

    out = pl.pallas_call(
        _tc3_body,
        out_shape=jax.ShapeDtypeStruct((N, OUT), _f32),
    )(acc2, den2, B1sel, bias2.reshape(1, OUT))

    return out
